# baseline (speedup 1.0000x reference)
import numpy as np
import jax
import jax.numpy as jnp
from jax import lax
from itertools import combinations
from jax.experimental import pallas as pl

_N = 24
_NPAIR = _N * (_N - 1) // 2
_L = _N + _NPAIR
_NR = _NPAIR + (_N * (_N - 1) * (_N - 2)) // 6
_LP, _RP = 384, 2304
_NLAYERS = 11
_F32 = jnp.float32
_BF = jnp.bfloat16


def _build_consts():
    pidx = {c: i for i, c in enumerate(combinations(range(_N), 2))}
    C = []
    for (u, v) in combinations(range(_N), 2):
        C.append([u, v, _N + pidx[(u, v)]])
    for (i, j, k) in combinations(range(_N), 3):
        C.append([_N + pidx[(i, j)], _N + pidx[(i, k)], _N + pidx[(j, k)]])
    C = np.array(C, np.int64)

    OH = np.zeros((3, _RP, _LP), np.float32)
    for jj in range(3):
        OH[jj, np.arange(_NR), C[:, jj]] = 1.0

    AEmb = np.zeros((_LP, _N), np.float32)
    AEmb[np.arange(_N), np.arange(_N)] = 1.0
    ZEmb = np.zeros((_LP, _N * (_N - 1)), np.float32)
    for (u, v) in combinations(range(_N), 2):
        ZEmb[_N + pidx[(u, v)], u * (_N - 1) + v - 1] = 1.0

    PSel = np.zeros((576, _LP), np.float32)
    PRC = np.zeros((576, _N), np.float32)
    RowSum = np.zeros((_N, 576), np.float32)
    q = 0
    for i in range(_N):
        for j in range(_N):
            if i == j:
                continue
            PSel[q, _N + pidx[(min(i, j), max(i, j))]] = 1.0
            PRC[q, i] += 1.0
            PRC[q, j] += 1.0
            q += 1
    for i in range(_N):
        RowSum[i, i * (_N - 1):(i + 1) * (_N - 1)] = 1.0
    return OH, AEmb, ZEmb, PSel, PRC, RowSum


_OH, _AEMB, _ZEMB, _PSEL, _PRC, _ROWSUM = _build_consts()
_MULT = (1.0, 1.0, 21.0)


def _split2(x):
    xh = x.astype(_BF)
    xl = (x - xh.astype(_F32)).astype(_BF)
    return xh, xl


def _dot(a, b):
    return jnp.dot(a, b, preferred_element_type=_F32)


def _dot_nn_t(a, b):
    return lax.dot_general(a, b, (((0,), (0,)), ((), ())),
                           preferred_element_type=_F32)


def _dot_nt(a, b):
    return lax.dot_general(a, b, (((1,), (1,)), ((), ())),
                           preferred_element_type=_F32)


def _sel(sbf, x):
    xh, xl = _split2(x)
    return _dot(sbf, xh) + _dot(sbf, xl)


def _gmm(a, b):
    ah, al = _split2(a)
    bh, bl = _split2(b)
    return _dot(ah, bh) + (_dot(ah, bl) + _dot(al, bh))


def _gmm_nt(a, b):
    ah, al = _split2(a)
    bh, bl = _split2(b)
    return _dot_nt(ah, bh) + (_dot_nt(ah, bl) + _dot_nt(al, bh))


def _ln_relu(x, g, b):
    mu = jnp.mean(x, axis=-1, keepdims=True)
    xc = x - mu
    var = jnp.mean(xc * xc, axis=-1, keepdims=True)
    return jax.nn.relu(xc * jax.lax.rsqrt(var + 1e-5) * g + b)


def _body(refs):
    (act_ref, inter_ref, know_ref,
     oh0_ref, oh1_ref, oh2_ref, aemb_ref, zemb_ref,
     psel_ref, prc_ref, rowsum_ref,
     aw1, ab1, aw2, ab2, alng, alnb,
     iw1, ib1, iw2, ib2, ilng, ilnb,
     fw1, fb1, fw2, fb2, icw, icb,
     ewA, ewB, eb_ref, lam_ref,
     fgnn_refs, oa_ref, oi_ref) = refs

    OHb = (oh0_ref[...], oh1_ref[...], oh2_ref[...])

    a = jax.nn.relu(_gmm(act_ref[...], aw1[...]) + ab1[...])
    a = _gmm(a, aw2[...]) + ab2[...]
    a = _ln_relu(a, alng[...], alnb[...])

    s = jax.nn.relu(_gmm(inter_ref[...], iw1[...]) + ib1[...])
    s = _gmm(s, iw2[...]) + ib2[...]
    s = _ln_relu(s, ilng[...], ilnb[...])

    Lf = _sel(aemb_ref[...], a) + _sel(zemb_ref[...], s)
    Rf = (_sel(OHb[0], Lf) + _sel(OHb[1], Lf) + _sel(OHb[2], Lf)) * (1.0 / 3.0)

    eb = eb_ref[...]
    PA_L, PB_L = _gmm(Lf, ewA[...]), _gmm(Lf, ewB[...])
    PA_R, PB_R = _gmm(Rf, ewA[...]), _gmm(Rf, ewB[...])
    WR = jax.nn.relu(jnp.concatenate(
        [PA_R + _sel(OHb[jj], PB_L) + eb for jj in range(3)], axis=1))
    WL = jax.nn.relu(jnp.concatenate(
        [_sel(OHb[jj], PA_L) + PB_R + eb for jj in range(3)], axis=1))

    def term3(xh, xl, wh, wl):
        return _dot(xh, wh) + (_dot(xh, wl) + _dot(xl, wh))

    hL, hR = Lf, Rf
    for li in range(_NLAYERS):
        wsh, wsl, wmh, wml, wer, b = fgnn_refs[6 * li:6 * li + 6]
        we = wer[...]
        DRh = DRl = DLh = DLl = None
        for jj in range(3):
            vr = jax.nn.relu(jnp.sum(WR[:, 16 * jj:16 * jj + 16] * we,
                                     axis=1, keepdims=True)) * (_MULT[jj] / 23.0)
            vl = jax.nn.relu(jnp.sum(WL[:, 16 * jj:16 * jj + 16] * we,
                                     axis=1, keepdims=True)) * (1.0 / 23.0)
            vrh, vrl = _split2(vr)
            vlh, vll = _split2(vl)
            tRh, tRl = vrh * OHb[jj], vrl * OHb[jj]
            tLh, tLl = vlh * OHb[jj], vll * OHb[jj]
            DRh = tRh if DRh is None else DRh + tRh
            DRl = tRl if DRl is None else DRl + tRl
            DLh = tLh if DLh is None else DLh + tLh
            DLl = tLl if DLl is None else DLl + tLl
        hLh, hLl = _split2(hL)
        hRh, hRl = _split2(hR)
        P = term3(hLh, hLl, wmh[...], wml[...])
        Ph, Pl = _split2(P)
        msgL = _dot_nn_t(DLh, hRh) + (_dot_nn_t(DLh, hRl) + _dot_nn_t(DLl, hRh))
        mLh, mLl = _split2(msgL)
        bb = b[...]
        hR = jax.nn.relu(term3(hRh, hRl, wsh[...], wsl[...])
                         + (_dot(DRh, Ph) + (_dot(DRh, Pl) + _dot(DRl, Ph)))
                         + bb)
        hL = jax.nn.relu(term3(hLh, hLl, wsh[...], wsl[...])
                         + term3(mLh, mLl, wmh[...], wml[...]) + bb)

    actn = hL[0:_N, :]
    act_out = _gmm(jax.nn.relu(_gmm(a + actn, fw1[...]) + fb1[...]), fw2[...]) \
        + fb2[...]
    inter_out = _gmm(_sel(psel_ref[...], hL), icw[...]) + icb[...]

    K = know_ref[...]
    lh = lam_ref[0, 0]
    lg = lam_ref[0, 1]
    act, inter = act_out, inter_out
    for _ in range(3):
        qa = jax.nn.softmax(act, axis=-1)
        qi = jax.nn.softmax(inter, axis=-1)
        Qs = _sel(rowsum_ref[...], qi)
        act = act_out + lh * _gmm_nt(Qs, K)
        inter = inter_out + lg * _sel(prc_ref[...], _gmm(qa, K))
    oa_ref[...] = act
    oi_ref[...] = inter[0:552, :]


def _run(act_score, inter_score, knowledge, flat_params):
    n_in = 11 + 22 + 6 * _NLAYERS

    def body(*refs):
        _body((*refs[0:11],
               *refs[11:33],
               refs[33:33 + 6 * _NLAYERS],
               refs[n_in], refs[n_in + 1]))

    bf = lambda x: jnp.asarray(x, _BF)
    consts = (bf(_OH[0]), bf(_OH[1]), bf(_OH[2]),
              bf(_AEMB), bf(_ZEMB),
              bf(_PSEL), bf(_PRC), bf(_ROWSUM))
    return pl.pallas_call(
        body,
        out_shape=[jax.ShapeDtypeStruct((_N, 32), _F32),
                   jax.ShapeDtypeStruct((552, 2), _F32)],
    )(act_score, inter_score, knowledge, *consts, *flat_params)


def kernel(act_score, inter_score, n_person, knowledge, params):
    p = params
    r2 = lambda x: x.reshape(1, -1)
    flat = [p['aff_w1'], r2(p['aff_b1']), p['aff_w2'], r2(p['aff_b2']),
            r2(p['aff_ln_g']), r2(p['aff_ln_b']),
            p['iff_w1'], r2(p['iff_b1']), p['iff_w2'], r2(p['iff_b2']),
            r2(p['iff_ln_g']), r2(p['iff_ln_b']),
            p['afc_w1'], r2(p['afc_b1']), p['afc_w2'], r2(p['afc_b2']),
            p['ifc_w'], r2(p['ifc_b']),
            p['edge_w'][:128, :], p['edge_w'][128:, :], r2(p['edge_b']),
            jnp.stack([p['lambda_h'][0], p['lambda_g'][0]]).reshape(1, 2)]
    for (ws, wm, we, b) in p['fgnn']:
        wsh = ws.astype(jnp.bfloat16)
        wsl = (ws - wsh.astype(jnp.float32)).astype(jnp.bfloat16)
        wmh = wm.astype(jnp.bfloat16)
        wml = (wm - wmh.astype(jnp.float32)).astype(jnp.bfloat16)
        flat.extend([wsh, wsl, wmh, wml, r2(we), r2(b)])
    act, inter = _run(act_score, inter_score, knowledge, flat)
    return act, inter
